# Initial kernel scaffold; baseline (speedup 1.0000x reference)
import functools

import jax
import jax.numpy as jnp
from jax import lax
from jax.experimental import pallas as pl
from jax.experimental.pallas import tpu as pltpu
from jax.experimental.pallas import tpu_sc as plsc

_N_USER = 50000
_N = 100000
_D = 32
_DH = 16
_E = 1600000
_B = 4096
_L = 8
_BL = _B * _L

_NP = 102400
_CH = 128
_Q = 8
_RT = 12544
_EP = _RT * _CH
_BLK = 1024
_NPB = _NP // _BLK

_f32 = jnp.float32


def _mesh():
    return plsc.VectorSubcoreMesh(core_axis_name="c", subcore_axis_name="s")


def _sc_degree(dst2d):
    rpw = _RT // 32
    grp = rpw // _Q
    npt = _NP // 16

    @functools.partial(
        pl.kernel,
        out_type=jax.ShapeDtypeStruct((2 * _NP,), _f32),
        mesh=_mesh(),
        scratch_types=[
            pltpu.VMEM((_Q, _CH), jnp.int32),
            pltpu.VMEM((_CH,), _f32),
            pltpu.VMEM((npt,), _f32),
            pltpu.VMEM_SHARED((_NP,), _f32),
            pltpu.SemaphoreType.DMA,
        ],
    )
    def k(dst_h, out_h, didx, ones, obuf, dacc, sem):
        c = lax.axis_index("c")
        s = lax.axis_index("s")
        for i in range(_CH // 16):
            ones[pl.ds(i * 16, 16)] = jnp.ones((16,), _f32)

        @pl.loop(0, npt // 16)
        def _(i):
            obuf[pl.ds(i * 16, 16)] = jnp.zeros((16,), _f32)

        pltpu.sync_copy(obuf, dacc.at[pl.ds(s * npt, npt)])
        plsc.subcore_barrier()

        @pl.loop(0, grp)
        def _(g):
            base = (c * 16 + s) * rpw + g * _Q
            pltpu.sync_copy(dst_h.at[pl.ds(base, _Q)], didx)
            descs = [
                pltpu.async_copy(ones, dacc.at[didx.at[q]], sem, add=True)
                for q in range(_Q)
            ]
            for d in descs:
                d.wait()

        plsc.subcore_barrier()
        pltpu.sync_copy(dacc.at[pl.ds(s * npt, npt)], obuf)
        pltpu.sync_copy(obuf, out_h.at[pl.ds(c * _NP + s * npt, npt)])

    return k(dst2d)


def _sc_segsum(table, srcall, dst2d):
    rpt = _RT // 16
    grp = rpt // _Q
    npt = _NP // 16
    wb = 400

    @functools.partial(
        pl.kernel,
        out_type=jax.ShapeDtypeStruct((2 * _NP, _DH), _f32),
        mesh=_mesh(),
        scratch_types=[
            pltpu.VMEM((_Q, _CH), jnp.int32),
            pltpu.VMEM((_Q, _CH), jnp.int32),
            pltpu.VMEM((_Q, _CH, _DH), _f32),
            pltpu.VMEM((wb, _DH), _f32),
            pltpu.VMEM_SHARED((_NP, _DH), _f32),
            pltpu.SemaphoreType.DMA,
            pltpu.SemaphoreType.DMA,
        ],
    )
    def k(table_h, src_h, dst_h, out_h, sidx, didx, rows, obuf, acc, gsem, ssem):
        c = lax.axis_index("c")
        s = lax.axis_index("s")

        @pl.loop(0, wb)
        def _(i):
            obuf[i] = jnp.zeros((_DH,), _f32)

        @pl.loop(0, npt // wb)
        def _(kk):
            pltpu.sync_copy(obuf, acc.at[pl.ds(s * npt + kk * wb, wb)])

        plsc.subcore_barrier()

        @pl.loop(0, grp)
        def _(g):
            base = s * rpt + g * _Q
            pltpu.sync_copy(src_h.at[pl.ds(c * _RT + base, _Q)], sidx)
            pltpu.sync_copy(dst_h.at[pl.ds(base, _Q)], didx)
            gd = [
                pltpu.async_copy(table_h.at[sidx.at[q]], rows.at[q], gsem)
                for q in range(_Q)
            ]
            for d in gd:
                d.wait()
            sd = [
                pltpu.async_copy(rows.at[q], acc.at[didx.at[q]], ssem, add=True)
                for q in range(_Q)
            ]
            for d in sd:
                d.wait()

        plsc.subcore_barrier()

        @pl.loop(0, npt // wb)
        def _(kk):
            r0 = s * npt + kk * wb
            pltpu.sync_copy(acc.at[pl.ds(r0, wb)], obuf)
            pltpu.sync_copy(obuf, out_h.at[pl.ds(c * _NP + r0, wb)])

    return k(table, srcall, dst2d)


def _sc_gather(fc, ft, uix2d, gix2d):
    nch = _BL // _CH
    cpw = nch // 32

    @functools.partial(
        pl.kernel,
        out_type=[jax.ShapeDtypeStruct((_BL, _D), _f32) for _ in range(4)],
        mesh=_mesh(),
        scratch_types=[
            pltpu.VMEM((cpw, _CH), jnp.int32),
            pltpu.VMEM((cpw, _CH), jnp.int32),
            pltpu.VMEM((_CH, _D), _f32),
            pltpu.VMEM((_CH, _D), _f32),
            pltpu.VMEM((_CH, _D), _f32),
            pltpu.VMEM((_CH, _D), _f32),
            pltpu.SemaphoreType.DMA,
        ],
    )
    def k(fc_h, ft_h, u_h, g_h, uc_h, ic_h, ut_h, it_h,
          ubuf, gbuf, r1, r2, r3, r4, sem):
        c = lax.axis_index("c")
        s = lax.axis_index("s")
        w = s * 2 + c
        pltpu.sync_copy(u_h.at[pl.ds(w * cpw, cpw)], ubuf)
        pltpu.sync_copy(g_h.at[pl.ds(w * cpw, cpw)], gbuf)

        for q in range(cpw):
            off = (w * cpw + q) * _CH
            ds = [
                pltpu.async_copy(fc_h.at[ubuf.at[q]], r1, sem),
                pltpu.async_copy(fc_h.at[gbuf.at[q]], r2, sem),
                pltpu.async_copy(ft_h.at[ubuf.at[q]], r3, sem),
                pltpu.async_copy(ft_h.at[gbuf.at[q]], r4, sem),
            ]
            for d in ds:
                d.wait()
            pltpu.sync_copy(r1, uc_h.at[pl.ds(off, _CH)])
            pltpu.sync_copy(r2, ic_h.at[pl.ds(off, _CH)])
            pltpu.sync_copy(r3, ut_h.at[pl.ds(off, _CH)])
            pltpu.sync_copy(r4, it_h.at[pl.ds(off, _CH)])

    return k(fc, ft, uix2d, gix2d)


def _tc_scale(deg2, emb_p):
    def body(deg_ref, emb_ref, g1_ref, norm_ref):
        deg = deg_ref[:, 0:1] + deg_ref[:, 1:2]
        n = lax.rsqrt(jnp.maximum(deg, 1.0))
        g1_ref[...] = emb_ref[...] * n
        norm_ref[...] = n

    return pl.pallas_call(
        body,
        grid=(2, _NPB),
        in_specs=[
            pl.BlockSpec((_BLK, 2), lambda h, i: (i, 0)),
            pl.BlockSpec((_BLK, _DH), lambda h, i: (i, h)),
        ],
        out_specs=[
            pl.BlockSpec((_BLK, _DH), lambda h, i: (h * _NPB + i, 0)),
            pl.BlockSpec((_BLK, 1), lambda h, i: (i, 0)),
        ],
        out_shape=[
            jax.ShapeDtypeStruct((2 * _NP, _DH), _f32),
            jax.ShapeDtypeStruct((_NP, 1), _f32),
        ],
    )(deg2, emb_p)


def _tc_mid(a1, norm):
    def body(a1_ref, norm_ref, g2_ref, h1_ref):
        n = norm_ref[...]
        a = a1_ref[...]
        h1_ref[...] = a * n
        g2_ref[...] = a * n * n

    return pl.pallas_call(
        body,
        grid=(2, _NPB),
        in_specs=[
            pl.BlockSpec((_BLK, _DH), lambda h, i: (h * _NPB + i, 0)),
            pl.BlockSpec((_BLK, 1), lambda h, i: (i, 0)),
        ],
        out_specs=[
            pl.BlockSpec((_BLK, _DH), lambda h, i: (h * _NPB + i, 0)),
            pl.BlockSpec((_BLK, _DH), lambda h, i: (h * _NPB + i, 0)),
        ],
        out_shape=[
            jax.ShapeDtypeStruct((2 * _NP, _DH), _f32),
            jax.ShapeDtypeStruct((2 * _NP, _DH), _f32),
        ],
    )(a1, norm)


def _tc_combine(emb_p, h1, a2, norm):
    def body(emb_ref, h1_ref, a2_ref, norm_ref, f_ref):
        f_ref[...] = (emb_ref[...] + h1_ref[...]
                      + norm_ref[...] * a2_ref[...]) * (1.0 / 3.0)

    return pl.pallas_call(
        body,
        grid=(2, _NPB),
        in_specs=[
            pl.BlockSpec((_BLK, _DH), lambda h, i: (i, h)),
            pl.BlockSpec((_BLK, _DH), lambda h, i: (h * _NPB + i, 0)),
            pl.BlockSpec((_BLK, _DH), lambda h, i: (h * _NPB + i, 0)),
            pl.BlockSpec((_BLK, 1), lambda h, i: (i, 0)),
        ],
        out_specs=pl.BlockSpec((_BLK, _DH), lambda h, i: (i, h)),
        out_shape=jax.ShapeDtypeStruct((_NP, _D), _f32),
    )(emb_p, h1, a2, norm)


def _tc_final(uc, ic, ut, it, lab, selc):
    rb = 2048
    nblk = _BL // rb

    def body(uc_ref, ic_ref, ut_ref, it_ref, lab_ref, selc_ref, *outs):
        i = pl.program_id(0)
        y = lab_ref[...]
        sel_c = selc_ref[...]
        sel_t = 1.0 - sel_c

        s_c = jnp.sum(uc_ref[...] * ic_ref[...], axis=1, keepdims=True)
        s_t = jnp.sum(ut_ref[...] * it_ref[...], axis=1, keepdims=True)

        def bce(s):
            return (jnp.maximum(s, 0.0) - s * y
                    + jnp.log(1.0 + jnp.exp(-jnp.abs(s))))

        def dist(s):
            return jnp.abs(1.0 / (1.0 + jnp.exp(-s)) - y)

        du = uc_ref[...] - ut_ref[...]
        di = ic_ref[...] - it_ref[...]
        vals = (
            jnp.sum(sel_c * bce(s_c)),
            jnp.sum(sel_c * dist(s_c)),
            jnp.sum(sel_t * bce(s_t)),
            jnp.sum(sel_t * dist(s_t)),
            jnp.sum(du * du),
            jnp.sum(di * di),
            jnp.sum(sel_c),
        )
        for ref, v in zip(outs, vals):
            prev = jnp.where(i == 0, 0.0, ref[0, 0])
            ref[0, 0] = prev + v

    scalar_spec = pl.BlockSpec((1, 1), lambda i: (0, 0),
                               memory_space=pltpu.SMEM)
    return pl.pallas_call(
        body,
        grid=(nblk,),
        in_specs=[
            pl.BlockSpec((rb, _D), lambda i: (i, 0)),
            pl.BlockSpec((rb, _D), lambda i: (i, 0)),
            pl.BlockSpec((rb, _D), lambda i: (i, 0)),
            pl.BlockSpec((rb, _D), lambda i: (i, 0)),
            pl.BlockSpec((rb, 1), lambda i: (i, 0)),
            pl.BlockSpec((rb, 1), lambda i: (i, 0)),
        ],
        out_specs=[scalar_spec] * 7,
        out_shape=[jax.ShapeDtypeStruct((1, 1), _f32)] * 7,
    )(uc, ic, ut, it, lab, selc)


def kernel(emb_control, emb_treatment, user, item, label, mask,
           edge_index_control, edge_index_treatment):
    pad_n = _NP - _N
    embc = jnp.concatenate([emb_control, jnp.zeros((pad_n, _D), _f32)])
    embt = jnp.concatenate([emb_treatment, jnp.zeros((pad_n, _D), _f32)])

    def prep_edges(ei):
        pad_e = _EP - _E
        srcp = jnp.concatenate([ei[0], jnp.zeros((pad_e,), jnp.int32)])
        dstp = jnp.concatenate([ei[1], jnp.full((pad_e,), _N, jnp.int32)])
        srcall = jnp.concatenate([srcp, srcp + _NP]).reshape(2 * _RT, _CH)
        return srcall, dstp.reshape(_RT, _CH)

    sa_c, d2_c = prep_edges(edge_index_control)
    sa_t, d2_t = prep_edges(edge_index_treatment)

    uix = user.reshape(_BL // _CH, _CH)
    gix = (item + _N_USER).reshape(_BL // _CH, _CH)
    lab = label.reshape(_BL, 1)
    selc = jnp.broadcast_to(jnp.logical_not(mask)[:, None],
                            (_B, _L)).astype(_f32).reshape(_BL, 1)

    def propagate(emb_p, srcall, dst2d):
        degf = _sc_degree(dst2d)
        deg2 = jnp.stack([degf[:_NP], degf[_NP:]], axis=1)
        g1, norm = _tc_scale(deg2, emb_p)
        a1 = _sc_segsum(g1, srcall, dst2d)
        g2, h1 = _tc_mid(a1, norm)
        a2 = _sc_segsum(g2, srcall, dst2d)
        return _tc_combine(emb_p, h1, a2, norm)

    fc = propagate(embc, sa_c, d2_c)
    ft = propagate(embt, sa_t, d2_t)

    uc, ic, ut, it = _sc_gather(fc, ft, uix, gix)
    sums = _tc_final(uc, ic, ut, it, lab, selc)
    bce_c, dist_c, bce_t, dist_t, disc_u, disc_i, cnt_c = [
        x[0, 0] for x in sums
    ]
    cnt_t = float(_BL) - cnt_c
    nel = float(_BL * _D)
    control_loss = bce_c / cnt_c
    treatment_loss = bce_t / cnt_t
    discrepancy = disc_u / nel + disc_i / nel
    control_distance = dist_c / cnt_c
    treatment_distance = dist_t / cnt_t
    return (control_loss, treatment_loss, discrepancy,
            control_distance, treatment_distance)


# speedup vs baseline: 7.9728x; 7.9728x over previous
import functools

import jax
import jax.numpy as jnp
from jax import lax
from jax.experimental import pallas as pl
from jax.experimental.pallas import tpu as pltpu
from jax.experimental.pallas import tpu_sc as plsc

_N_USER = 50000
_N = 100000
_D = 32
_DH = 16
_E = 1600000
_B = 4096
_L = 8
_BL = _B * _L

_NP = 102400
_CH = 128
_Q = 8
_RT = 12544
_EP = _RT * _CH
_BLK = 1024
_NPB = _NP // _BLK

_f32 = jnp.float32


def _mesh():
    return plsc.VectorSubcoreMesh(core_axis_name="c", subcore_axis_name="s")


def _sc_degree(dst2d):
    rpw = _RT // 32
    grp = rpw // _Q
    npt = _NP // 16

    @functools.partial(
        pl.kernel,
        out_type=jax.ShapeDtypeStruct((2 * _NP,), _f32),
        mesh=_mesh(),
        compiler_params=pltpu.CompilerParams(use_tc_tiling_on_sc=False),
        scratch_types=[
            pltpu.VMEM((_Q, _CH), jnp.int32),
            pltpu.VMEM((_CH,), _f32),
            pltpu.VMEM((npt,), _f32),
            pltpu.VMEM_SHARED((_NP,), _f32),
            pltpu.SemaphoreType.DMA,
        ],
    )
    def k(dst_h, out_h, didx, ones, obuf, dacc, sem):
        c = lax.axis_index("c")
        s = lax.axis_index("s")
        for i in range(_CH // 16):
            ones[pl.ds(i * 16, 16)] = jnp.ones((16,), _f32)

        @pl.loop(0, npt // 16)
        def _(i):
            obuf[pl.ds(i * 16, 16)] = jnp.zeros((16,), _f32)

        pltpu.sync_copy(obuf, dacc.at[pl.ds(s * npt, npt)])
        plsc.subcore_barrier()

        @pl.loop(0, grp)
        def _(g):
            base = (c * 16 + s) * rpw + g * _Q
            pltpu.sync_copy(dst_h.at[pl.ds(base, _Q)], didx)
            descs = [
                pltpu.async_copy(ones, dacc.at[didx.at[q]], sem, add=True)
                for q in range(_Q)
            ]
            for d in descs:
                d.wait()

        plsc.subcore_barrier()
        pltpu.sync_copy(dacc.at[pl.ds(s * npt, npt)], obuf)
        pltpu.sync_copy(obuf, out_h.at[pl.ds(c * _NP + s * npt, npt)])

    return k(dst2d)


def _sc_segsum(table, srcall, dst2d):
    rpt = _RT // 16
    grp = rpt // _Q
    npt = _NP // 16
    wb = 400

    @functools.partial(
        pl.kernel,
        out_type=jax.ShapeDtypeStruct((2 * _NP, _DH), _f32),
        mesh=_mesh(),
        compiler_params=pltpu.CompilerParams(use_tc_tiling_on_sc=False),
        scratch_types=[
            pltpu.VMEM((_Q, _CH), jnp.int32),
            pltpu.VMEM((_Q, _CH), jnp.int32),
            pltpu.VMEM((_Q, _CH, _DH), _f32),
            pltpu.VMEM((wb, _DH), _f32),
            pltpu.VMEM_SHARED((_NP, _DH), _f32),
            pltpu.SemaphoreType.DMA,
            pltpu.SemaphoreType.DMA,
        ],
    )
    def k(table_h, src_h, dst_h, out_h, sidx, didx, rows, obuf, acc, gsem, ssem):
        c = lax.axis_index("c")
        s = lax.axis_index("s")

        @pl.loop(0, wb)
        def _(i):
            obuf[i] = jnp.zeros((_DH,), _f32)

        @pl.loop(0, npt // wb)
        def _(kk):
            pltpu.sync_copy(obuf, acc.at[pl.ds(s * npt + kk * wb, wb)])

        plsc.subcore_barrier()

        @pl.loop(0, grp)
        def _(g):
            base = s * rpt + g * _Q
            pltpu.sync_copy(src_h.at[pl.ds(c * _RT + base, _Q)], sidx)
            pltpu.sync_copy(dst_h.at[pl.ds(base, _Q)], didx)
            gd = [
                pltpu.async_copy(table_h.at[sidx.at[q]], rows.at[q], gsem)
                for q in range(_Q)
            ]
            for d in gd:
                d.wait()
            sd = [
                pltpu.async_copy(rows.at[q], acc.at[didx.at[q]], ssem, add=True)
                for q in range(_Q)
            ]
            for d in sd:
                d.wait()

        plsc.subcore_barrier()

        @pl.loop(0, npt // wb)
        def _(kk):
            r0 = s * npt + kk * wb
            pltpu.sync_copy(acc.at[pl.ds(r0, wb)], obuf)
            pltpu.sync_copy(obuf, out_h.at[pl.ds(c * _NP + r0, wb)])

    return k(table, srcall, dst2d)


def _sc_gather(fc, ft, uix2, gix2):
    nch = _BL // _CH
    cpw = nch // 32

    @functools.partial(
        pl.kernel,
        out_type=[jax.ShapeDtypeStruct((2 * _BL, _DH), _f32)
                  for _ in range(4)],
        mesh=_mesh(),
        compiler_params=pltpu.CompilerParams(use_tc_tiling_on_sc=False),
        scratch_types=[
            pltpu.VMEM((cpw, _CH), jnp.int32),
            pltpu.VMEM((cpw, _CH), jnp.int32),
            pltpu.VMEM((cpw, _CH), jnp.int32),
            pltpu.VMEM((cpw, _CH), jnp.int32),
            pltpu.VMEM((8, _CH, _DH), _f32),
            pltpu.SemaphoreType.DMA,
        ],
    )
    def k(fc_h, ft_h, u_h, g_h, uc_h, ic_h, ut_h, it_h,
          ulo, uhi, glo, ghi, rows, sem):
        c = lax.axis_index("c")
        s = lax.axis_index("s")
        w = s * 2 + c
        pltpu.sync_copy(u_h.at[pl.ds(w * cpw, cpw)], ulo)
        pltpu.sync_copy(u_h.at[pl.ds(nch + w * cpw, cpw)], uhi)
        pltpu.sync_copy(g_h.at[pl.ds(w * cpw, cpw)], glo)
        pltpu.sync_copy(g_h.at[pl.ds(nch + w * cpw, cpw)], ghi)

        for q in range(cpw):
            off = (w * cpw + q) * _CH
            ds = [
                pltpu.async_copy(fc_h.at[ulo.at[q]], rows.at[0], sem),
                pltpu.async_copy(fc_h.at[uhi.at[q]], rows.at[1], sem),
                pltpu.async_copy(fc_h.at[glo.at[q]], rows.at[2], sem),
                pltpu.async_copy(fc_h.at[ghi.at[q]], rows.at[3], sem),
                pltpu.async_copy(ft_h.at[ulo.at[q]], rows.at[4], sem),
                pltpu.async_copy(ft_h.at[uhi.at[q]], rows.at[5], sem),
                pltpu.async_copy(ft_h.at[glo.at[q]], rows.at[6], sem),
                pltpu.async_copy(ft_h.at[ghi.at[q]], rows.at[7], sem),
            ]
            for d in ds:
                d.wait()
            pltpu.sync_copy(rows.at[0], uc_h.at[pl.ds(off, _CH)])
            pltpu.sync_copy(rows.at[1], uc_h.at[pl.ds(_BL + off, _CH)])
            pltpu.sync_copy(rows.at[2], ic_h.at[pl.ds(off, _CH)])
            pltpu.sync_copy(rows.at[3], ic_h.at[pl.ds(_BL + off, _CH)])
            pltpu.sync_copy(rows.at[4], ut_h.at[pl.ds(off, _CH)])
            pltpu.sync_copy(rows.at[5], ut_h.at[pl.ds(_BL + off, _CH)])
            pltpu.sync_copy(rows.at[6], it_h.at[pl.ds(off, _CH)])
            pltpu.sync_copy(rows.at[7], it_h.at[pl.ds(_BL + off, _CH)])

    return k(fc, ft, uix2, gix2)


def _tc_scale(deg2, embs):
    def body(deg_ref, emb_ref, g1_ref, norm_ref):
        deg = deg_ref[:, 0:1] + deg_ref[:, 1:2]
        n = lax.rsqrt(jnp.maximum(deg, 1.0))
        g1_ref[...] = emb_ref[...] * n
        norm_ref[...] = n

    return pl.pallas_call(
        body,
        grid=(2, _NPB),
        in_specs=[
            pl.BlockSpec((_BLK, 2), lambda h, i: (i, 0)),
            pl.BlockSpec((_BLK, _DH), lambda h, i: (h * _NPB + i, 0)),
        ],
        out_specs=[
            pl.BlockSpec((_BLK, _DH), lambda h, i: (h * _NPB + i, 0)),
            pl.BlockSpec((_BLK, 1), lambda h, i: (i, 0)),
        ],
        out_shape=[
            jax.ShapeDtypeStruct((2 * _NP, _DH), _f32),
            jax.ShapeDtypeStruct((_NP, 1), _f32),
        ],
    )(deg2, embs)


def _tc_mid(a1, norm):
    def body(a1_ref, norm_ref, g2_ref, h1_ref):
        n = norm_ref[...]
        a = a1_ref[...]
        h1_ref[...] = a * n
        g2_ref[...] = a * n * n

    return pl.pallas_call(
        body,
        grid=(2, _NPB),
        in_specs=[
            pl.BlockSpec((_BLK, _DH), lambda h, i: (h * _NPB + i, 0)),
            pl.BlockSpec((_BLK, 1), lambda h, i: (i, 0)),
        ],
        out_specs=[
            pl.BlockSpec((_BLK, _DH), lambda h, i: (h * _NPB + i, 0)),
            pl.BlockSpec((_BLK, _DH), lambda h, i: (h * _NPB + i, 0)),
        ],
        out_shape=[
            jax.ShapeDtypeStruct((2 * _NP, _DH), _f32),
            jax.ShapeDtypeStruct((2 * _NP, _DH), _f32),
        ],
    )(a1, norm)


def _tc_combine(embs, h1, a2, norm):
    def body(emb_ref, h1_ref, a2_ref, norm_ref, f_ref):
        f_ref[...] = (emb_ref[...] + h1_ref[...]
                      + norm_ref[...] * a2_ref[...]) * (1.0 / 3.0)

    return pl.pallas_call(
        body,
        grid=(2, _NPB),
        in_specs=[
            pl.BlockSpec((_BLK, _DH), lambda h, i: (h * _NPB + i, 0)),
            pl.BlockSpec((_BLK, _DH), lambda h, i: (h * _NPB + i, 0)),
            pl.BlockSpec((_BLK, _DH), lambda h, i: (h * _NPB + i, 0)),
            pl.BlockSpec((_BLK, 1), lambda h, i: (i, 0)),
        ],
        out_specs=pl.BlockSpec((_BLK, _DH), lambda h, i: (h * _NPB + i, 0)),
        out_shape=jax.ShapeDtypeStruct((2 * _NP, _DH), _f32),
    )(embs, h1, a2, norm)


def _tc_final(uc, ic, ut, it, lab, selc):
    rb = 2048
    nblk = _BL // rb
    hoff = _BL // rb

    def body(ucl, uch, icl, ich, utl, uth, itl, ith,
             lab_ref, selc_ref, *outs):
        i = pl.program_id(0)
        y = lab_ref[...]
        sel_c = selc_ref[...]
        sel_t = 1.0 - sel_c

        s_c = jnp.sum(ucl[...] * icl[...] + uch[...] * ich[...],
                      axis=1, keepdims=True)
        s_t = jnp.sum(utl[...] * itl[...] + uth[...] * ith[...],
                      axis=1, keepdims=True)

        def bce(s):
            return (jnp.maximum(s, 0.0) - s * y
                    + jnp.log(1.0 + jnp.exp(-jnp.abs(s))))

        def dist(s):
            return jnp.abs(1.0 / (1.0 + jnp.exp(-s)) - y)

        dul = ucl[...] - utl[...]
        duh = uch[...] - uth[...]
        dil = icl[...] - itl[...]
        dih = ich[...] - ith[...]
        vals = (
            jnp.sum(sel_c * bce(s_c)),
            jnp.sum(sel_c * dist(s_c)),
            jnp.sum(sel_t * bce(s_t)),
            jnp.sum(sel_t * dist(s_t)),
            jnp.sum(dul * dul) + jnp.sum(duh * duh),
            jnp.sum(dil * dil) + jnp.sum(dih * dih),
            jnp.sum(sel_c),
        )
        for ref, v in zip(outs, vals):
            prev = jnp.where(i == 0, 0.0, ref[0, 0])
            ref[0, 0] = prev + v

    lo = pl.BlockSpec((rb, _DH), lambda i: (i, 0))
    hi = pl.BlockSpec((rb, _DH), lambda i: (hoff + i, 0))
    scalar_spec = pl.BlockSpec((1, 1), lambda i: (0, 0),
                               memory_space=pltpu.SMEM)
    return pl.pallas_call(
        body,
        grid=(nblk,),
        in_specs=[lo, hi, lo, hi, lo, hi, lo, hi,
                  pl.BlockSpec((rb, 1), lambda i: (i, 0)),
                  pl.BlockSpec((rb, 1), lambda i: (i, 0))],
        out_specs=[scalar_spec] * 7,
        out_shape=[jax.ShapeDtypeStruct((1, 1), _f32)] * 7,
    )(uc, uc, ic, ic, ut, ut, it, it, lab, selc)


def kernel(emb_control, emb_treatment, user, item, label, mask,
           edge_index_control, edge_index_treatment):
    pad_n = _NP - _N

    def split_emb(e):
        ep = jnp.concatenate([e, jnp.zeros((pad_n, _D), _f32)])
        return jnp.concatenate([ep[:, :_DH], ep[:, _DH:]], axis=0)

    embc = split_emb(emb_control)
    embt = split_emb(emb_treatment)

    def prep_edges(ei):
        pad_e = _EP - _E
        srcp = jnp.concatenate([ei[0], jnp.zeros((pad_e,), jnp.int32)])
        dstp = jnp.concatenate([ei[1], jnp.full((pad_e,), _N, jnp.int32)])
        srcall = jnp.concatenate([srcp, srcp + _NP]).reshape(2 * _RT, _CH)
        return srcall, dstp.reshape(_RT, _CH)

    sa_c, d2_c = prep_edges(edge_index_control)
    sa_t, d2_t = prep_edges(edge_index_treatment)

    uflat = user.reshape(_BL)
    gflat = (item + _N_USER).reshape(_BL)
    uix2 = jnp.concatenate([uflat, uflat + _NP]).reshape(2 * _BL // _CH, _CH)
    gix2 = jnp.concatenate([gflat, gflat + _NP]).reshape(2 * _BL // _CH, _CH)
    lab = label.reshape(_BL, 1)
    selc = jnp.broadcast_to(jnp.logical_not(mask)[:, None],
                            (_B, _L)).astype(_f32).reshape(_BL, 1)

    def propagate(embs, srcall, dst2d):
        degf = _sc_degree(dst2d)
        deg2 = jnp.stack([degf[:_NP], degf[_NP:]], axis=1)
        g1, norm = _tc_scale(deg2, embs)
        a1 = _sc_segsum(g1, srcall, dst2d)
        g2, h1 = _tc_mid(a1, norm)
        a2 = _sc_segsum(g2, srcall, dst2d)
        return _tc_combine(embs, h1, a2, norm)

    fc = propagate(embc, sa_c, d2_c)
    ft = propagate(embt, sa_t, d2_t)

    uc, ic, ut, it = _sc_gather(fc, ft, uix2, gix2)
    sums = _tc_final(uc, ic, ut, it, lab, selc)
    bce_c, dist_c, bce_t, dist_t, disc_u, disc_i, cnt_c = [
        x[0, 0] for x in sums
    ]
    cnt_t = float(_BL) - cnt_c
    nel = float(_BL * _D)
    control_loss = bce_c / cnt_c
    treatment_loss = bce_t / cnt_t
    discrepancy = disc_u / nel + disc_i / nel
    control_distance = dist_c / cnt_c
    treatment_distance = dist_t / cnt_t
    return (control_loss, treatment_loss, discrepancy,
            control_distance, treatment_distance)


# speedup vs baseline: 10.6544x; 1.3363x over previous
import functools

import jax
import jax.numpy as jnp
from jax import lax
from jax.experimental import pallas as pl
from jax.experimental.pallas import tpu as pltpu
from jax.experimental.pallas import tpu_sc as plsc

_N_USER = 50000
_N = 100000
_D = 32
_DH = 16
_E = 1600000
_B = 4096
_L = 8
_BL = _B * _L

_NP = 102400
_CH = 128
_Q = 8
_RT = 12544
_EP = _RT * _CH

_VH = _NP * _DH // 128
_BR = 512
_NBT = _VH // _BR

_f32 = jnp.float32


def _mesh():
    return plsc.VectorSubcoreMesh(core_axis_name="c", subcore_axis_name="s")


def _sc_degree(dstall):
    rpt = _RT // 16
    grp = rpt // _Q
    npt = _NP // 16

    @functools.partial(
        pl.kernel,
        out_type=jax.ShapeDtypeStruct((2 * _NP,), _f32),
        mesh=_mesh(),
        compiler_params=pltpu.CompilerParams(use_tc_tiling_on_sc=False),
        scratch_types=[
            pltpu.VMEM((_Q, _CH), jnp.int32),
            pltpu.VMEM((_CH,), _f32),
            pltpu.VMEM((npt,), _f32),
            pltpu.VMEM_SHARED((_NP,), _f32),
            pltpu.SemaphoreType.DMA,
        ],
    )
    def k(dst_h, out_h, didx, ones, obuf, dacc, sem):
        c = lax.axis_index("c")
        s = lax.axis_index("s")
        for i in range(_CH // 16):
            ones[pl.ds(i * 16, 16)] = jnp.ones((16,), _f32)

        @pl.loop(0, npt // 16)
        def _(i):
            obuf[pl.ds(i * 16, 16)] = jnp.zeros((16,), _f32)

        pltpu.sync_copy(obuf, dacc.at[pl.ds(s * npt, npt)])
        plsc.subcore_barrier()

        @pl.loop(0, grp)
        def _(g):
            base = (c * 16 + s) * rpt + g * _Q
            pltpu.sync_copy(dst_h.at[pl.ds(base, _Q)], didx)
            descs = [
                pltpu.async_copy(ones, dacc.at[didx.at[q]], sem, add=True)
                for q in range(_Q)
            ]
            for d in descs:
                d.wait()

        plsc.subcore_barrier()
        pltpu.sync_copy(dacc.at[pl.ds(s * npt, npt)], obuf)
        pltpu.sync_copy(obuf, out_h.at[pl.ds(c * _NP + s * npt, npt)])

    return k(dstall)


def _sc_segsum(table, srcall, dst2d):
    rpt = _RT // 16
    grp = rpt // _Q
    npt = _NP // 16
    wb = 400

    @functools.partial(
        pl.kernel,
        out_type=jax.ShapeDtypeStruct((2 * _NP, _DH), _f32),
        mesh=_mesh(),
        compiler_params=pltpu.CompilerParams(use_tc_tiling_on_sc=False),
        scratch_types=[
            pltpu.VMEM((_Q, _CH), jnp.int32),
            pltpu.VMEM((_Q, _CH), jnp.int32),
            pltpu.VMEM((_Q, _CH, _DH), _f32),
            pltpu.VMEM((wb, _DH), _f32),
            pltpu.VMEM_SHARED((_NP, _DH), _f32),
            pltpu.SemaphoreType.DMA,
            pltpu.SemaphoreType.DMA,
        ],
    )
    def k(table_h, src_h, dst_h, out_h, sidx, didx, rows, zbuf, acc,
          gsem, ssem):
        c = lax.axis_index("c")
        s = lax.axis_index("s")

        @pl.loop(0, wb)
        def _(i):
            zbuf[i] = jnp.zeros((_DH,), _f32)

        @pl.loop(0, npt // wb)
        def _(kk):
            pltpu.sync_copy(zbuf, acc.at[pl.ds(s * npt + kk * wb, wb)])

        plsc.subcore_barrier()

        @pl.loop(0, grp)
        def _(g):
            base = s * rpt + g * _Q
            pltpu.sync_copy(src_h.at[pl.ds(c * _RT + base, _Q)], sidx)
            pltpu.sync_copy(dst_h.at[pl.ds(base, _Q)], didx)
            gd = [
                pltpu.async_copy(table_h.at[sidx.at[q]], rows.at[q], gsem)
                for q in range(_Q)
            ]
            sd = []
            for q in range(_Q // 2):
                gd[q].wait()
                sd.append(pltpu.async_copy(
                    rows.at[q], acc.at[didx.at[q]], ssem, add=True))
            for q in range(_Q // 2, _Q):
                gd[q].wait()
                sd.append(pltpu.async_copy(
                    rows.at[q], acc.at[didx.at[q]], ssem, add=True))
            for d in sd:
                d.wait()

        plsc.subcore_barrier()

        @pl.loop(0, npt // wb)
        def _(kk):
            r0 = s * npt + kk * wb
            pltpu.sync_copy(acc.at[pl.ds(r0, wb)], zbuf)
            pltpu.sync_copy(zbuf, out_h.at[pl.ds(c * _NP + r0, wb)])

    return k(table, srcall, dst2d)


def _sc_gather(fc, ft, uix2, gix2):
    bs = _BL // 32

    @functools.partial(
        pl.kernel,
        out_type=[jax.ShapeDtypeStruct((2 * _BL, _DH), _f32)
                  for _ in range(4)],
        mesh=_mesh(),
        compiler_params=pltpu.CompilerParams(use_tc_tiling_on_sc=False),
        scratch_types=[
            pltpu.VMEM((bs,), jnp.int32),
            pltpu.VMEM((bs,), jnp.int32),
            pltpu.VMEM((bs,), jnp.int32),
            pltpu.VMEM((bs,), jnp.int32),
            pltpu.VMEM((bs, _DH), _f32),
            pltpu.VMEM((bs, _DH), _f32),
            pltpu.SemaphoreType.DMA,
            pltpu.SemaphoreType.DMA,
        ],
    )
    def k(fc_h, ft_h, u_h, g_h, uc_h, ic_h, ut_h, it_h,
          ulo, uhi, glo, ghi, rowsa, rowsb, sema, semb):
        c = lax.axis_index("c")
        s = lax.axis_index("s")
        w = s * 2 + c
        pltpu.sync_copy(u_h.at[pl.ds(w * bs, bs)], ulo)
        pltpu.sync_copy(u_h.at[pl.ds(_BL + w * bs, bs)], uhi)
        pltpu.sync_copy(g_h.at[pl.ds(w * bs, bs)], glo)
        pltpu.sync_copy(g_h.at[pl.ds(_BL + w * bs, bs)], ghi)

        combos = [
            (fc_h, ulo, uc_h, 0), (fc_h, uhi, uc_h, _BL),
            (fc_h, glo, ic_h, 0), (fc_h, ghi, ic_h, _BL),
            (ft_h, ulo, ut_h, 0), (ft_h, uhi, ut_h, _BL),
            (ft_h, glo, it_h, 0), (ft_h, ghi, it_h, _BL),
        ]
        pend = None
        for j, (tab, ib, ob, hoff) in enumerate(combos):
            buf = rowsa if j % 2 == 0 else rowsb
            sem = sema if j % 2 == 0 else semb
            d = pltpu.async_copy(tab.at[ib], buf, sem)
            if pend is not None:
                pd, pbuf, pob, phoff = pend
                pd.wait()
                pltpu.sync_copy(pbuf, pob.at[pl.ds(phoff + w * bs, bs)])
            pend = (d, buf, ob, hoff)
        pd, pbuf, pob, phoff = pend
        pd.wait()
        pltpu.sync_copy(pbuf, pob.at[pl.ds(phoff + w * bs, bs)])

    return k(fc, ft, uix2, gix2)


def _tc_prep(degt):
    blk = 2048

    def body(deg_ref, nwc_ref, nwt_ref):
        n = lax.rsqrt(jnp.maximum(deg_ref[...], 1.0))
        one = jnp.ones((1, _DH), _f32)
        nwc_ref[...] = n[:, 0:1] * one
        nwt_ref[...] = n[:, 1:2] * one

    return pl.pallas_call(
        body,
        grid=(_NP // blk,),
        in_specs=[pl.BlockSpec((blk, 2), lambda i: (i, 0))],
        out_specs=[pl.BlockSpec((blk, _DH), lambda i: (i, 0))] * 2,
        out_shape=[jax.ShapeDtypeStruct((_NP, _DH), _f32)] * 2,
    )(degt)


def _wide_specs():
    tab = pl.BlockSpec((_BR, 128), lambda h, j: (h * _NBT + j, 0))
    nw = pl.BlockSpec((_BR, 128), lambda h, j: (j, 0))
    return tab, nw


def _tc_scale(embs_c, nw_c, embs_t, nw_t):
    tab, nw = _wide_specs()

    def body(ec, nc, et, nt, g1c, g1t):
        g1c[...] = ec[...] * nc[...]
        g1t[...] = et[...] * nt[...]

    return pl.pallas_call(
        body,
        grid=(2, _NBT),
        in_specs=[tab, nw, tab, nw],
        out_specs=[tab, tab],
        out_shape=[jax.ShapeDtypeStruct((2 * _VH, 128), _f32)] * 2,
    )(embs_c, nw_c, embs_t, nw_t)


def _tc_mid(a1_c, nw_c, a1_t, nw_t):
    tab, nw = _wide_specs()

    def body(ac, nc, at_, nt, g2c, h1c, g2t, h1t):
        hc = ac[...] * nc[...]
        h1c[...] = hc
        g2c[...] = hc * nc[...]
        ht = at_[...] * nt[...]
        h1t[...] = ht
        g2t[...] = ht * nt[...]

    return pl.pallas_call(
        body,
        grid=(2, _NBT),
        in_specs=[tab, nw, tab, nw],
        out_specs=[tab, tab, tab, tab],
        out_shape=[jax.ShapeDtypeStruct((2 * _VH, 128), _f32)] * 4,
    )(a1_c, nw_c, a1_t, nw_t)


def _tc_combine(embs_c, h1_c, a2_c, nw_c, embs_t, h1_t, a2_t, nw_t):
    tab, nw = _wide_specs()

    def body(ec, hc, ac, nc, et, ht, at_, nt, fc, ft):
        fc[...] = (ec[...] + hc[...] + nc[...] * ac[...]) * (1.0 / 3.0)
        ft[...] = (et[...] + ht[...] + nt[...] * at_[...]) * (1.0 / 3.0)

    return pl.pallas_call(
        body,
        grid=(2, _NBT),
        in_specs=[tab, tab, tab, nw, tab, tab, tab, nw],
        out_specs=[tab, tab],
        out_shape=[jax.ShapeDtypeStruct((2 * _VH, 128), _f32)] * 2,
    )(embs_c, h1_c, a2_c, nw_c, embs_t, h1_t, a2_t, nw_t)


def _tc_final(uc, ic, ut, it, lab, selc):
    rb = 2048
    nblk = _BL // rb
    hoff = _BL // rb

    def body(ucl, uch, icl, ich, utl, uth, itl, ith,
             lab_ref, selc_ref, *outs):
        i = pl.program_id(0)
        y = lab_ref[...]
        sel_c = selc_ref[...]
        sel_t = 1.0 - sel_c

        s_c = jnp.sum(ucl[...] * icl[...] + uch[...] * ich[...],
                      axis=1, keepdims=True)
        s_t = jnp.sum(utl[...] * itl[...] + uth[...] * ith[...],
                      axis=1, keepdims=True)

        def bce(s):
            return (jnp.maximum(s, 0.0) - s * y
                    + jnp.log(1.0 + jnp.exp(-jnp.abs(s))))

        def dist(s):
            return jnp.abs(1.0 / (1.0 + jnp.exp(-s)) - y)

        dul = ucl[...] - utl[...]
        duh = uch[...] - uth[...]
        dil = icl[...] - itl[...]
        dih = ich[...] - ith[...]
        vals = (
            jnp.sum(sel_c * bce(s_c)),
            jnp.sum(sel_c * dist(s_c)),
            jnp.sum(sel_t * bce(s_t)),
            jnp.sum(sel_t * dist(s_t)),
            jnp.sum(dul * dul) + jnp.sum(duh * duh),
            jnp.sum(dil * dil) + jnp.sum(dih * dih),
            jnp.sum(sel_c),
        )
        for ref, v in zip(outs, vals):
            prev = jnp.where(i == 0, 0.0, ref[0, 0])
            ref[0, 0] = prev + v

    lo = pl.BlockSpec((rb, _DH), lambda i: (i, 0))
    hi = pl.BlockSpec((rb, _DH), lambda i: (hoff + i, 0))
    scalar_spec = pl.BlockSpec((1, 1), lambda i: (0, 0),
                               memory_space=pltpu.SMEM)
    return pl.pallas_call(
        body,
        grid=(nblk,),
        in_specs=[lo, hi, lo, hi, lo, hi, lo, hi,
                  pl.BlockSpec((rb, 1), lambda i: (i, 0)),
                  pl.BlockSpec((rb, 1), lambda i: (i, 0))],
        out_specs=[scalar_spec] * 7,
        out_shape=[jax.ShapeDtypeStruct((1, 1), _f32)] * 7,
    )(uc, uc, ic, ic, ut, ut, it, it, lab, selc)


def kernel(emb_control, emb_treatment, user, item, label, mask,
           edge_index_control, edge_index_treatment):
    pad_n = _NP - _N

    def split_emb(e):
        ep = jnp.concatenate([e, jnp.zeros((pad_n, _D), _f32)])
        s = jnp.concatenate([ep[:, :_DH], ep[:, _DH:]], axis=0)
        return s.reshape(2 * _VH, 128)

    embc = split_emb(emb_control)
    embt = split_emb(emb_treatment)

    def prep_edges(ei):
        pad_e = _EP - _E
        srcp = jnp.concatenate([ei[0], jnp.zeros((pad_e,), jnp.int32)])
        dstp = jnp.concatenate([ei[1], jnp.full((pad_e,), _N, jnp.int32)])
        srcall = jnp.concatenate([srcp, srcp + _NP]).reshape(2 * _RT, _CH)
        return srcall, dstp.reshape(_RT, _CH)

    sa_c, d2_c = prep_edges(edge_index_control)
    sa_t, d2_t = prep_edges(edge_index_treatment)
    dstall = jnp.concatenate([d2_c, d2_t], axis=0)

    uflat = user.reshape(_BL)
    gflat = (item + _N_USER).reshape(_BL)
    uix2 = jnp.concatenate([uflat, uflat + _NP])
    gix2 = jnp.concatenate([gflat, gflat + _NP])
    lab = label.reshape(_BL, 1)
    selc = jnp.broadcast_to(jnp.logical_not(mask)[:, None],
                            (_B, _L)).astype(_f32).reshape(_BL, 1)

    degall = _sc_degree(dstall)
    degt = jnp.stack([degall[:_NP], degall[_NP:]], axis=1)
    nw_c, nw_t = _tc_prep(degt)
    nw_c = nw_c.reshape(_VH, 128)
    nw_t = nw_t.reshape(_VH, 128)

    def as_tbl(x):
        return x.reshape(2 * _NP, _DH)

    def as_flat(x):
        return x.reshape(2 * _VH, 128)

    g1_c, g1_t = _tc_scale(embc, nw_c, embt, nw_t)
    a1_c = as_flat(_sc_segsum(as_tbl(g1_c), sa_c, d2_c))
    a1_t = as_flat(_sc_segsum(as_tbl(g1_t), sa_t, d2_t))
    g2_c, h1_c, g2_t, h1_t = _tc_mid(a1_c, nw_c, a1_t, nw_t)
    a2_c = as_flat(_sc_segsum(as_tbl(g2_c), sa_c, d2_c))
    a2_t = as_flat(_sc_segsum(as_tbl(g2_t), sa_t, d2_t))
    fc, ft = _tc_combine(embc, h1_c, a2_c, nw_c, embt, h1_t, a2_t, nw_t)

    uc, ic, ut, it = _sc_gather(as_tbl(fc), as_tbl(ft), uix2, gix2)
    sums = _tc_final(uc, ic, ut, it, lab, selc)
    bce_c, dist_c, bce_t, dist_t, disc_u, disc_i, cnt_c = [
        x[0, 0] for x in sums
    ]
    cnt_t = float(_BL) - cnt_c
    nel = float(_BL * _D)
    control_loss = bce_c / cnt_c
    treatment_loss = bce_t / cnt_t
    discrepancy = disc_u / nel + disc_i / nel
    control_distance = dist_c / cnt_c
    treatment_distance = dist_t / cnt_t
    return (control_loss, treatment_loss, discrepancy,
            control_distance, treatment_distance)
